# baseline (speedup 1.0000x reference)
import functools

import jax
import jax.numpy as jnp
from jax import lax
from jax.experimental import pallas as pl
from jax.experimental.pallas import tpu as pltpu
from jax.experimental.pallas import tpu_sc as plsc

N = 100000
D = 128
S = 64
NW = 32
LANES = 16
NJ = D // LANES

ROWS = 3120
EXTRA_TILES = 20
SUB = 104
NSUB = ROWS // SUB
SEARCH_STEPS = 12

_mesh = lambda: plsc.VectorSubcoreMesh(core_axis_name="c", subcore_axis_name="s")


def _tile_base(wid):
    return wid * ROWS + jnp.minimum(wid, EXTRA_TILES) * 8


def _tile_count(wid):
    return ROWS + jnp.where(wid < EXTRA_TILES, 8, 0)


def _find_starts(idxbuf, starts, count):
    s_first = idxbuf[pl.ds(0, LANES)][0]
    s_last = idxbuf[pl.ds(count - LANES, LANES)][LANES - 1]
    starts[0] = 0
    starts[S] = count

    def seg(s, carry):
        in_span = jnp.logical_and(s > s_first, s <= s_last)

        @pl.when(in_span)
        def _():
            def probe(_, lohi):
                lo, hi = lohi
                mid = lax.div(lo + hi, 2)
                v = idxbuf[pl.ds(mid, LANES)][0]
                small = v < s
                return (jnp.where(small, mid + 1, lo), jnp.where(small, hi, mid))

            lo, _ = lax.fori_loop(0, SEARCH_STEPS, probe, (jnp.int32(0), count))
            starts[s] = lo

        @pl.when(jnp.logical_not(in_span))
        def _():
            starts[s] = jnp.where(s <= s_first, 0, count)

        return carry

    lax.fori_loop(1, S, seg, 0)


def _seg_span(idxbuf, chunk_lo, chunk_hi):
    s_first = idxbuf[pl.ds(chunk_lo, LANES)][0]
    s_last = idxbuf[pl.ds(chunk_hi - LANES, LANES)][LANES - 1]
    return s_first, s_last


@functools.partial(
    pl.kernel,
    out_type=(
        jax.ShapeDtypeStruct((NW, S, D), jnp.float32),
        jax.ShapeDtypeStruct((NW, S, LANES), jnp.float32),
    ),
    mesh=_mesh(),
    scratch_types=(
        pltpu.VMEM((4, SUB, D), jnp.float32),
        pltpu.VMEM((ROWS + 32,), jnp.int32),
        pltpu.VMEM((S, D), jnp.float32),
        pltpu.VMEM((S, LANES), jnp.float32),
        pltpu.SMEM((S + 1,), jnp.int32),
        pltpu.SemaphoreType.DMA((4,)),
    ),
)
def _sc_segment_sums(node_hbm, idx_hbm, part_hbm, cnt_hbm, buf, idxbuf, acc, cnt,
                     starts, sem):
    wid = lax.axis_index("c") * 16 + lax.axis_index("s")
    base = _tile_base(wid)
    count = _tile_count(wid)
    has_extra = wid < EXTRA_TILES

    zeros = jnp.zeros((LANES,), jnp.float32)

    def in_copy(c):
        return pltpu.make_async_copy(
            node_hbm.at[pl.ds(base + c * SUB, SUB)], buf.at[c % 4], sem.at[c % 4])

    for c in range(4):
        in_copy(c).start()

    pltpu.sync_copy(idx_hbm.at[pl.ds(base, ROWS)], idxbuf.at[pl.ds(0, ROWS)])

    @pl.when(has_extra)
    def _():
        pltpu.sync_copy(idx_hbm.at[pl.ds(base + ROWS, 8)], idxbuf.at[pl.ds(ROWS, 8)])

    _find_starts(idxbuf, starts, count)

    def zrow(i, carry):
        for j in range(NJ):
            acc[i, pl.ds(j * LANES, LANES)] = zeros
        c = (starts[i + 1] - starts[i]).astype(jnp.float32)
        cnt[i] = jnp.broadcast_to(c, (LANES,))
        return carry

    lax.fori_loop(0, S, zrow, 0)

    def accum_chunk(bref, chunk_lo, chunk_hi):
        s_first, s_last = _seg_span(idxbuf, chunk_lo, chunk_hi)

        def seg(s, carry):
            lo = jnp.maximum(starts[s], chunk_lo)
            hi = jnp.minimum(starts[s + 1], chunk_hi)

            def row(r, accv):
                b = r - chunk_lo
                return tuple(accv[j] + bref[b, pl.ds(j * LANES, LANES)]
                             for j in range(NJ))

            sums = plsc.parallel_loop(lo, hi, unroll=8,
                                      carry=(zeros,) * NJ)(row)
            for j in range(NJ):
                plsc.addupdate(acc.at[s, pl.ds(j * LANES, LANES)], sums[j])
            return carry

        lax.fori_loop(s_first, s_last + 1, seg, 0)

    for c in range(NSUB):
        in_copy(c).wait()
        accum_chunk(buf.at[c % 4], c * SUB, c * SUB + SUB)
        if c + 4 < NSUB:
            in_copy(c + 4).start()

    @pl.when(has_extra)
    def _():
        pltpu.sync_copy(node_hbm.at[pl.ds(base + ROWS, 8)], buf.at[0, pl.ds(0, 8)])
        accum_chunk(buf.at[0], ROWS, count)

    pltpu.sync_copy(acc, part_hbm.at[wid])
    pltpu.sync_copy(cnt, cnt_hbm.at[wid])


def _mlp_body(part_ref, cnt_ref, w1_ref, b1_ref, w2_ref, b2_ref, mod_ref):
    sums = jnp.sum(part_ref[...], axis=0)
    cnts = jnp.sum(cnt_ref[...], axis=0)[:, 0:1]
    mean = sums / jnp.maximum(cnts, 1.0)
    h = lax.dot_general(mean, w1_ref[...], (((1,), (1,)), ((), ())),
                        preferred_element_type=jnp.float32) + b1_ref[...]
    h = jnp.maximum(h, 0.0)
    z = lax.dot_general(h, w2_ref[...], (((1,), (1,)), ((), ())),
                        preferred_element_type=jnp.float32) + b2_ref[...]
    mod_ref[...] = 1.0 / (1.0 + jnp.exp(-z))


_tc_mlp = pl.pallas_call(
    _mlp_body,
    out_shape=jax.ShapeDtypeStruct((S, D), jnp.float32),
)


@functools.partial(
    pl.kernel,
    out_type=jax.ShapeDtypeStruct((N, D), jnp.float32),
    mesh=_mesh(),
    scratch_types=(
        pltpu.VMEM((3, SUB, D), jnp.float32),
        pltpu.VMEM((4, SUB, D), jnp.float32),
        pltpu.VMEM((ROWS + 32,), jnp.int32),
        pltpu.VMEM((S, D), jnp.float32),
        pltpu.SMEM((S + 1,), jnp.int32),
        pltpu.SemaphoreType.DMA((3,)),
        pltpu.SemaphoreType.DMA((4,)),
    ),
)
def _sc_apply(node_hbm, idx_hbm, mod_hbm, out_hbm, bin_, bout, idxbuf, modv,
              starts, semin, semout):
    wid = lax.axis_index("c") * 16 + lax.axis_index("s")
    base = _tile_base(wid)
    count = _tile_count(wid)
    has_extra = wid < EXTRA_TILES

    def in_copy(c):
        return pltpu.make_async_copy(
            node_hbm.at[pl.ds(base + c * SUB, SUB)], bin_.at[c % 3],
            semin.at[c % 3])

    def out_copy(c):
        return pltpu.make_async_copy(
            bout.at[c % 4], out_hbm.at[pl.ds(base + c * SUB, SUB)],
            semout.at[c % 4])

    in_copy(0).start()
    in_copy(1).start()
    in_copy(2).start()

    pltpu.sync_copy(mod_hbm, modv)
    pltpu.sync_copy(idx_hbm.at[pl.ds(base, ROWS)], idxbuf.at[pl.ds(0, ROWS)])

    @pl.when(has_extra)
    def _():
        pltpu.sync_copy(idx_hbm.at[pl.ds(base + ROWS, 8)], idxbuf.at[pl.ds(ROWS, 8)])

    _find_starts(idxbuf, starts, count)

    def apply_chunk(bi, bo, chunk_lo, chunk_hi):
        s_first, s_last = _seg_span(idxbuf, chunk_lo, chunk_hi)

        def seg(s, carry):
            lo = jnp.maximum(starts[s], chunk_lo)
            hi = jnp.minimum(starts[s + 1], chunk_hi)
            mod = tuple(modv[s, pl.ds(j * LANES, LANES)] for j in range(NJ))

            @plsc.parallel_loop(lo, hi, unroll=8)
            def row(r):
                b = r - chunk_lo
                for j in range(NJ):
                    bo[b, pl.ds(j * LANES, LANES)] = (
                        bi[b, pl.ds(j * LANES, LANES)] * mod[j])
            return carry

        lax.fori_loop(s_first, s_last + 1, seg, 0)

    for c in range(NSUB):
        in_copy(c).wait()
        if c >= 4:
            out_copy(c - 4).wait()
        apply_chunk(bin_.at[c % 3], bout.at[c % 4], c * SUB, c * SUB + SUB)
        out_copy(c).start()
        if c + 3 < NSUB:
            in_copy(c + 3).start()

    for c in range(NSUB - 4, NSUB):
        out_copy(c).wait()

    @pl.when(has_extra)
    def _():
        pltpu.sync_copy(node_hbm.at[pl.ds(base + ROWS, 8)], bin_.at[0, pl.ds(0, 8)])
        apply_chunk(bin_.at[0], bout.at[0], ROWS, count)
        pltpu.sync_copy(bout.at[0, pl.ds(0, 8)], out_hbm.at[pl.ds(base + ROWS, 8)])


def kernel(node_feat, batch_idx, W1, b1, W2, b2):
    idx32 = batch_idx.astype(jnp.int32)
    partials, counts = _sc_segment_sums(node_feat, idx32)
    mod = _tc_mlp(partials, counts, W1, b1.reshape(1, D), W2, b2.reshape(1, D))
    return _sc_apply(node_feat, idx32, mod)


# speedup vs baseline: 1.0563x; 1.0563x over previous
import functools

import jax
import jax.numpy as jnp
from jax import lax
from jax.experimental import pallas as pl
from jax.experimental.pallas import tpu as pltpu
from jax.experimental.pallas import tpu_sc as plsc

N = 100000
D = 128
S = 64
NW = 32
LANES = 16
NJ = D // LANES

ROWS = 3120
EXTRA_TILES = 20
SUB = 208
NSUB = ROWS // SUB
SEARCH_STEPS = 12

_mesh = lambda: plsc.VectorSubcoreMesh(core_axis_name="c", subcore_axis_name="s")


def _tile_base(wid):
    return wid * ROWS + jnp.minimum(wid, EXTRA_TILES) * 8


def _tile_count(wid):
    return ROWS + jnp.where(wid < EXTRA_TILES, 8, 0)


def _find_starts(idxbuf, starts, count):
    s_first = idxbuf[pl.ds(0, LANES)][0]
    s_last = idxbuf[pl.ds(count - LANES, LANES)][LANES - 1]
    starts[0] = 0
    starts[S] = count

    def seg(s, carry):
        in_span = jnp.logical_and(s > s_first, s <= s_last)

        @pl.when(in_span)
        def _():
            def probe(_, lohi):
                lo, hi = lohi
                mid = lax.div(lo + hi, 2)
                v = idxbuf[pl.ds(mid, LANES)][0]
                small = v < s
                return (jnp.where(small, mid + 1, lo), jnp.where(small, hi, mid))

            lo, _ = lax.fori_loop(0, SEARCH_STEPS, probe, (jnp.int32(0), count))
            starts[s] = lo

        @pl.when(jnp.logical_not(in_span))
        def _():
            starts[s] = jnp.where(s <= s_first, 0, count)

        return carry

    lax.fori_loop(1, S, seg, 0)


def _seg_span(idxbuf, chunk_lo, chunk_hi):
    s_first = idxbuf[pl.ds(chunk_lo, LANES)][0]
    s_last = idxbuf[pl.ds(chunk_hi - LANES, LANES)][LANES - 1]
    return s_first, s_last


@functools.partial(
    pl.kernel,
    out_type=(
        jax.ShapeDtypeStruct((NW, S, D), jnp.float32),
        jax.ShapeDtypeStruct((NW, S, LANES), jnp.float32),
    ),
    mesh=_mesh(),
    scratch_types=(
        pltpu.VMEM((4, SUB, D), jnp.float32),
        pltpu.VMEM((ROWS + 32,), jnp.int32),
        pltpu.VMEM((S, D), jnp.float32),
        pltpu.VMEM((S, LANES), jnp.float32),
        pltpu.SMEM((S + 1,), jnp.int32),
        pltpu.SemaphoreType.DMA((4,)),
    ),
)
def _sc_segment_sums(node_hbm, idx_hbm, part_hbm, cnt_hbm, buf, idxbuf, acc, cnt,
                     starts, sem):
    wid = lax.axis_index("c") * 16 + lax.axis_index("s")
    base = _tile_base(wid)
    count = _tile_count(wid)
    has_extra = wid < EXTRA_TILES

    zeros = jnp.zeros((LANES,), jnp.float32)

    def in_copy(c):
        return pltpu.make_async_copy(
            node_hbm.at[pl.ds(base + c * SUB, SUB)], buf.at[c % 4], sem.at[c % 4])

    pltpu.sync_copy(idx_hbm.at[pl.ds(base, ROWS)], idxbuf.at[pl.ds(0, ROWS)])

    for c in range(4):
        in_copy(c).start()

    @pl.when(has_extra)
    def _():
        pltpu.sync_copy(idx_hbm.at[pl.ds(base + ROWS, 8)], idxbuf.at[pl.ds(ROWS, 8)])

    _find_starts(idxbuf, starts, count)

    def zrow(i, carry):
        for j in range(NJ):
            acc[i, pl.ds(j * LANES, LANES)] = zeros
        c = (starts[i + 1] - starts[i]).astype(jnp.float32)
        cnt[i] = jnp.broadcast_to(c, (LANES,))
        return carry

    lax.fori_loop(0, S, zrow, 0)

    def accum_chunk(bref, chunk_lo, chunk_hi):
        s_first, s_last = _seg_span(idxbuf, chunk_lo, chunk_hi)

        def seg(s, carry):
            lo = jnp.maximum(starts[s], chunk_lo)
            hi = jnp.minimum(starts[s + 1], chunk_hi)

            def row(r, accv):
                b = r - chunk_lo
                return tuple(accv[j] + bref[b, pl.ds(j * LANES, LANES)]
                             for j in range(NJ))

            sums = plsc.parallel_loop(lo, hi, unroll=8,
                                      carry=(zeros,) * NJ)(row)
            for j in range(NJ):
                plsc.addupdate(acc.at[s, pl.ds(j * LANES, LANES)], sums[j])
            return carry

        lax.fori_loop(s_first, s_last + 1, seg, 0)

    for c in range(NSUB):
        in_copy(c).wait()
        accum_chunk(buf.at[c % 4], c * SUB, c * SUB + SUB)
        if c + 4 < NSUB:
            in_copy(c + 4).start()

    @pl.when(has_extra)
    def _():
        pltpu.sync_copy(node_hbm.at[pl.ds(base + ROWS, 8)], buf.at[0, pl.ds(0, 8)])
        accum_chunk(buf.at[0], ROWS, count)

    pltpu.sync_copy(acc, part_hbm.at[wid])
    pltpu.sync_copy(cnt, cnt_hbm.at[wid])


def _mlp_body(part_ref, cnt_ref, w1_ref, b1_ref, w2_ref, b2_ref, mod_ref):
    sums = jnp.sum(part_ref[...], axis=0)
    cnts = jnp.sum(cnt_ref[...], axis=0)[:, 0:1]
    mean = sums / jnp.maximum(cnts, 1.0)
    h = lax.dot_general(mean, w1_ref[...], (((1,), (1,)), ((), ())),
                        preferred_element_type=jnp.float32) + b1_ref[...]
    h = jnp.maximum(h, 0.0)
    z = lax.dot_general(h, w2_ref[...], (((1,), (1,)), ((), ())),
                        preferred_element_type=jnp.float32) + b2_ref[...]
    mod_ref[...] = 1.0 / (1.0 + jnp.exp(-z))


_tc_mlp = pl.pallas_call(
    _mlp_body,
    out_shape=jax.ShapeDtypeStruct((S, D), jnp.float32),
)


@functools.partial(
    pl.kernel,
    out_type=jax.ShapeDtypeStruct((N, D), jnp.float32),
    mesh=_mesh(),
    scratch_types=(
        pltpu.VMEM((2, SUB, D), jnp.float32),
        pltpu.VMEM((2, SUB, D), jnp.float32),
        pltpu.VMEM((ROWS + 32,), jnp.int32),
        pltpu.VMEM((S, D), jnp.float32),
        pltpu.SMEM((S + 1,), jnp.int32),
        pltpu.SemaphoreType.DMA((2,)),
        pltpu.SemaphoreType.DMA((2,)),
    ),
)
def _sc_apply(node_hbm, idx_hbm, mod_hbm, out_hbm, bin_, bout, idxbuf, modv,
              starts, semin, semout):
    wid = lax.axis_index("c") * 16 + lax.axis_index("s")
    base = _tile_base(wid)
    count = _tile_count(wid)
    has_extra = wid < EXTRA_TILES

    def in_copy(c):
        return pltpu.make_async_copy(
            node_hbm.at[pl.ds(base + c * SUB, SUB)], bin_.at[c % 2],
            semin.at[c % 2])

    def out_copy(c):
        return pltpu.make_async_copy(
            bout.at[c % 2], out_hbm.at[pl.ds(base + c * SUB, SUB)],
            semout.at[c % 2])

    pltpu.sync_copy(idx_hbm.at[pl.ds(base, ROWS)], idxbuf.at[pl.ds(0, ROWS)])

    in_copy(0).start()
    in_copy(1).start()

    pltpu.sync_copy(mod_hbm, modv)

    @pl.when(has_extra)
    def _():
        pltpu.sync_copy(idx_hbm.at[pl.ds(base + ROWS, 8)], idxbuf.at[pl.ds(ROWS, 8)])

    _find_starts(idxbuf, starts, count)

    def apply_chunk(bi, bo, chunk_lo, chunk_hi):
        s_first, s_last = _seg_span(idxbuf, chunk_lo, chunk_hi)

        def seg(s, carry):
            lo = jnp.maximum(starts[s], chunk_lo)
            hi = jnp.minimum(starts[s + 1], chunk_hi)
            mod = tuple(modv[s, pl.ds(j * LANES, LANES)] for j in range(NJ))

            @plsc.parallel_loop(lo, hi, unroll=8)
            def row(r):
                b = r - chunk_lo
                for j in range(NJ):
                    bo[b, pl.ds(j * LANES, LANES)] = (
                        bi[b, pl.ds(j * LANES, LANES)] * mod[j])
            return carry

        lax.fori_loop(s_first, s_last + 1, seg, 0)

    for c in range(NSUB):
        in_copy(c).wait()
        if c >= 2:
            out_copy(c - 2).wait()
        apply_chunk(bin_.at[c % 2], bout.at[c % 2], c * SUB, c * SUB + SUB)
        out_copy(c).start()
        if c + 2 < NSUB:
            in_copy(c + 2).start()

    out_copy(NSUB - 2).wait()
    out_copy(NSUB - 1).wait()

    @pl.when(has_extra)
    def _():
        pltpu.sync_copy(node_hbm.at[pl.ds(base + ROWS, 8)], bin_.at[0, pl.ds(0, 8)])
        apply_chunk(bin_.at[0], bout.at[0], ROWS, count)
        pltpu.sync_copy(bout.at[0, pl.ds(0, 8)], out_hbm.at[pl.ds(base + ROWS, 8)])


def kernel(node_feat, batch_idx, W1, b1, W2, b2):
    idx32 = batch_idx.astype(jnp.int32)
    partials, counts = _sc_segment_sums(node_feat, idx32)
    mod = _tc_mlp(partials, counts, W1, b1.reshape(1, D), W2, b2.reshape(1, D))
    return _sc_apply(node_feat, idx32, mod)
